# baseline (speedup 1.0000x reference)
import functools

import jax
import jax.numpy as jnp
from jax import lax
from jax.experimental import pallas as pl
from jax.experimental.pallas import tpu as pltpu
from jax.experimental.pallas import tpu_sc as plsc


IDX_CHUNK = 128


def _make_gather_pool(V, D, B, C, DP):
    info = plsc.get_sparse_core_info()
    NC, NS, L = info.num_cores, info.num_subcores, info.num_lanes
    NW = NC * NS
    assert B % NW == 0 and D % L == 0
    b_per_w = B // NW
    n_idx = b_per_w * C
    assert n_idx % IDX_CHUNK == 0
    n_chunks = n_idx // IDX_CHUNK
    mesh = plsc.VectorSubcoreMesh(core_axis_name="c", subcore_axis_name="s")

    @functools.partial(
        pl.kernel,
        mesh=mesh,
        out_type=jax.ShapeDtypeStruct((B, D), jnp.float32),
        scratch_types=[
            pltpu.VMEM((n_idx,), jnp.int32),
            pltpu.VMEM((n_idx, DP), jnp.float32),
            pltpu.VMEM((b_per_w, D), jnp.float32),
            pltpu.SemaphoreType.DMA,
        ],
    )
    def gather_pool(idx_hbm, table_hbm, out_hbm, idx_v, rows_v, pooled_v, sem):
        wid = lax.axis_index("s") * NC + lax.axis_index("c")
        pltpu.sync_copy(idx_hbm.at[pl.ds(wid * n_idx, n_idx)], idx_v)
        copies = [
            pltpu.async_copy(
                table_hbm.at[idx_v.at[pl.ds(j * IDX_CHUNK, IDX_CHUNK)]],
                rows_v.at[pl.ds(j * IDX_CHUNK, IDX_CHUNK)],
                sem,
            )
            for j in range(n_chunks)
        ]
        for cp in copies:
            cp.wait()
        inv = jnp.full((L,), 1.0 / C, jnp.float32)

        def row_body(b, carry):
            r0 = b * C
            for d in range(D // L):
                acc = rows_v[r0, pl.ds(d * L, L)]
                for c in range(1, C):
                    acc = acc + rows_v[r0 + c, pl.ds(d * L, L)]
                pooled_v[b, pl.ds(d * L, L)] = acc * inv
            return carry

        lax.fori_loop(0, b_per_w, row_body, 0)
        pltpu.sync_copy(pooled_v, out_hbm.at[pl.ds(wid * b_per_w, b_per_w)])

    return gather_pool


def _make_dense_lsm(B, D, V, bt):
    def body(p_ref, w_ref, b_ref, o_ref):
        logits = jnp.dot(
            p_ref[...].astype(jnp.bfloat16),
            w_ref[...],
            preferred_element_type=jnp.float32,
        )
        logits = logits + b_ref[...]
        m = jnp.max(logits, axis=1, keepdims=True)
        s = logits - m
        lse = jnp.log(jnp.sum(jnp.exp(s), axis=1, keepdims=True))
        o_ref[...] = s - lse

    return pl.pallas_call(
        body,
        grid=(B // bt,),
        in_specs=[
            pl.BlockSpec((bt, D), lambda i: (i, 0)),
            pl.BlockSpec((D, V), lambda i: (0, 0)),
            pl.BlockSpec((1, V), lambda i: (0, 0)),
        ],
        out_specs=pl.BlockSpec((bt, V), lambda i: (i, 0)),
        out_shape=jax.ShapeDtypeStruct((B, V), jnp.float32),
    )


def kernel(contexts, emb_weight, lin_weight, lin_bias):
    B, C = contexts.shape
    V, D = emb_weight.shape
    idx = contexts.reshape(B * C).astype(jnp.int32)
    DP = 128
    table = jnp.pad(emb_weight, ((0, 0), (0, DP - D)))
    pooled = _make_gather_pool(V, D, B, C, DP)(idx, table)
    w_t = lin_weight.T.astype(jnp.bfloat16)
    bias2 = lin_bias.reshape(1, V)
    return _make_dense_lsm(B, D, V, 32)(pooled, w_t, bias2)
